# Initial kernel scaffold; baseline (speedup 1.0000x reference)
import functools

import jax
import jax.numpy as jnp
from jax import lax
from jax.experimental import pallas as pl
from jax.experimental.pallas import tpu as pltpu
from jax.experimental.pallas import tpu_sc as plsc

_NC = 2
_NS = 16
_NW = _NC * _NS
_L = 16

_B = 16


def _pick_chunk(epw):
    for cand in range(128, 7, -8):
        if epw % cand == 0:
            return cand
    raise ValueError(f"no valid chunk for {epw}")


@functools.lru_cache(maxsize=None)
def _make_agg(n, d, e):
    assert e % _NW == 0 and n % _NS == 0 and d % _L == 0
    epw = e // _NW
    ch = _pick_chunk(epw)
    nch = epw // ch
    rpt = n // _NS
    zr = 1
    for cand in range(32, 0, -1):
        if rpt % cand == 0:
            zr = cand
            break

    def body(h_hbm, src_hbm, dst_hbm, out_hbm, srcv, dstv, rows, zbuf, acc):
        c = lax.axis_index("c")
        s = lax.axis_index("s")
        wid = c * _NS + s

        for rr in range(zr):
            for jj in range(d // _L):
                zbuf[rr, pl.ds(jj * _L, _L)] = jnp.zeros((_L,), jnp.float32)
        row0 = s * rpt

        @pl.loop(0, rpt // zr)
        def _(i):
            pltpu.sync_copy(zbuf, acc.at[pl.ds(row0 + i * zr, zr), :])

        plsc.subcore_barrier()

        eb = wid * epw

        @pl.loop(0, nch)
        def _(g):
            off = eb + g * ch
            pltpu.sync_copy(src_hbm.at[pl.ds(off, ch)], srcv)
            pltpu.sync_copy(dst_hbm.at[pl.ds(off, ch)], dstv)
            pltpu.sync_copy(h_hbm.at[srcv], rows)
            pltpu.sync_copy(rows, acc.at[dstv], add=True)

        plsc.subcore_barrier()
        pltpu.sync_copy(acc.at[pl.ds(row0, rpt), :],
                        out_hbm.at[c, pl.ds(row0, rpt), :])

    return pl.kernel(
        body,
        out_type=jax.ShapeDtypeStruct((_NC, n, d), jnp.float32),
        mesh=plsc.VectorSubcoreMesh(core_axis_name="c", subcore_axis_name="s"),
        scratch_types=[
            pltpu.VMEM((ch,), jnp.int32),
            pltpu.VMEM((ch,), jnp.int32),
            pltpu.VMEM((ch, d), jnp.float32),
            pltpu.VMEM((zr, d), jnp.float32),
            pltpu.VMEM_SHARED((n, d), jnp.float32),
        ],
    )


@functools.lru_cache(maxsize=None)
def _make_deg(n, e):
    assert e % _NW == 0 and n % _L == 0
    epw = e // _NW
    ch = _pick_chunk(epw)
    nch = epw // ch

    def body(dst_hbm, out_hbm, dstv, hist):
        c = lax.axis_index("c")
        s = lax.axis_index("s")
        wid = c * _NS + s

        @pl.loop(0, n // _L)
        def _(i):
            hist[pl.ds(i * _L, _L)] = jnp.zeros((_L,), jnp.float32)

        eb = wid * epw
        ones = jnp.ones((_L,), jnp.float32)

        @pl.loop(0, nch)
        def _(g):
            pltpu.sync_copy(dst_hbm.at[pl.ds(eb + g * ch, ch)], dstv)
            for j in range(ch // _L):
                idx = dstv[pl.ds(j * _L, _L)]
                plsc.addupdate_scatter(hist, [idx], ones)

        pltpu.sync_copy(hist, out_hbm.at[wid])

    return pl.kernel(
        body,
        out_type=jax.ShapeDtypeStruct((_NW, n), jnp.float32),
        mesh=plsc.VectorSubcoreMesh(core_axis_name="c", subcore_axis_name="s"),
        scratch_types=[
            pltpu.VMEM((ch,), jnp.int32),
            pltpu.VMEM((n,), jnp.float32),
        ],
    )


def _prep_body(parts_ref, r_ref):
    deg = jnp.sum(parts_ref[...], axis=0)
    r_ref[...] = (1.0 / jnp.maximum(deg, 1.0))[:, None]


@functools.lru_cache(maxsize=None)
def _make_prep(n):
    return pl.pallas_call(
        _prep_body,
        out_shape=jax.ShapeDtypeStruct((n, 1), jnp.float32),
    )


def _dense_body(relu, p_ref, r_ref, wt_ref, b_ref, o_ref):
    h = (p_ref[0] + p_ref[1]) * r_ref[...]
    y = jnp.dot(h, wt_ref[...], preferred_element_type=jnp.float32,
                precision=lax.Precision.HIGHEST) + b_ref[...]
    o_ref[...] = jnp.maximum(y, 0.0) if relu else y


@functools.lru_cache(maxsize=None)
def _make_dense(n, d, rb, relu):
    grid = (n // rb,)
    return pl.pallas_call(
        functools.partial(_dense_body, relu),
        grid=grid,
        in_specs=[
            pl.BlockSpec((_NC, rb, d), lambda g: (0, g, 0)),
            pl.BlockSpec((rb, 1), lambda g: (g, 0)),
            pl.BlockSpec((d, d), lambda g: (0, 0)),
            pl.BlockSpec((1, d), lambda g: (0, 0)),
        ],
        out_specs=pl.BlockSpec((rb, d), lambda g: (g, 0)),
        out_shape=jax.ShapeDtypeStruct((n, d), jnp.float32),
    )


def _densepool_body(p_ref, r_ref, wt_ref, b_ref, bid_ref, o_ref):
    g = pl.program_id(0)
    h = (p_ref[0] + p_ref[1]) * r_ref[...]
    y = jnp.dot(h, wt_ref[...], preferred_element_type=jnp.float32,
                precision=lax.Precision.HIGHEST) + b_ref[...]

    @pl.when(g == 0)
    def _():
        o_ref[...] = jnp.full(o_ref.shape, -jnp.inf, jnp.float32)

    bid = bid_ref[0, 0, :]
    for k in range(_B):
        mk = (bid == k)[:, None]
        mx = jnp.max(jnp.where(mk, y, -jnp.inf), axis=0)
        o_ref[k, :] = jnp.maximum(o_ref[k, :], mx)


@functools.lru_cache(maxsize=None)
def _make_densepool(n, d, rb):
    nb = n // rb
    return pl.pallas_call(
        _densepool_body,
        grid=(nb,),
        in_specs=[
            pl.BlockSpec((_NC, rb, d), lambda g: (0, g, 0)),
            pl.BlockSpec((rb, 1), lambda g: (g, 0)),
            pl.BlockSpec((d, d), lambda g: (0, 0)),
            pl.BlockSpec((1, d), lambda g: (0, 0)),
            pl.BlockSpec((1, 1, rb), lambda g: (g, 0, 0)),
        ],
        out_specs=pl.BlockSpec((_B, d), lambda g: (0, 0)),
        out_shape=jax.ShapeDtypeStruct((_B, d), jnp.float32),
    )


def _heads_body(pool_ref, w1, b1, w2, b2, wo, bo, v1, c1, v2, c2, vo, co,
                q1_ref, q2_ref):
    p = pool_ref[...]

    def mlp(wa, ba, wb, bb, wc, bc):
        a = jnp.maximum(jnp.dot(p, wa[...], preferred_element_type=jnp.float32,
                                precision=lax.Precision.HIGHEST) + ba[...], 0.0)
        a = jnp.maximum(jnp.dot(a, wb[...], preferred_element_type=jnp.float32,
                                precision=lax.Precision.HIGHEST) + bb[...], 0.0)
        return jnp.dot(a, wc[...], preferred_element_type=jnp.float32,
                       precision=lax.Precision.HIGHEST) + bc[...]

    q1_ref[...] = mlp(w1, b1, w2, b2, wo, bo)
    q2_ref[...] = mlp(v1, c1, v2, c2, vo, co)


@functools.lru_cache(maxsize=None)
def _make_heads(d):
    return pl.pallas_call(
        _heads_body,
        out_shape=(jax.ShapeDtypeStruct((_B, 1), jnp.float32),
                   jax.ShapeDtypeStruct((_B, 1), jnp.float32)),
    )


def kernel(x, action, edge_index, batch,
           W_g0, b_g0, W_g1, b_g1, W_g2, b_g2,
           q1_W, q1_b, q1_2_W, q1_2_b, q1_out_W, q1_out_b,
           q2_W, q2_b, q2_2_W, q2_2_b, q2_out_W, q2_out_b):
    n = x.shape[0]
    e = edge_index.shape[1]
    d = W_g0.shape[0]
    src = edge_index[0]
    dst = edge_index[1]
    h0 = jnp.concatenate([x, action], axis=1)

    agg = _make_agg(n, d, e)
    rb = 1000
    dense_r = _make_dense(n, d, rb, True)

    degp = _make_deg(n, e)(dst)
    r = _make_prep(n)(degp)

    p = agg(h0, src, dst)
    h1 = dense_r(p, r, W_g0.T, b_g0.reshape(1, d))
    p = agg(h1, src, dst)
    h2 = dense_r(p, r, W_g1.T, b_g1.reshape(1, d))
    p = agg(h2, src, dst)
    pooled = _make_densepool(n, d, rb)(
        p, r, W_g2.T, b_g2.reshape(1, d), batch.reshape(n // rb, 1, rb))

    q1, q2 = _make_heads(d)(
        pooled,
        q1_W.T, q1_b.reshape(1, d), q1_2_W.T, q1_2_b.reshape(1, d),
        q1_out_W.T, q1_out_b.reshape(1, 1),
        q2_W.T, q2_b.reshape(1, d), q2_2_W.T, q2_2_b.reshape(1, d),
        q2_out_W.T, q2_out_b.reshape(1, 1))
    return (q1, q2)


# speedup vs baseline: 2.5335x; 2.5335x over previous
import functools

import jax
import jax.numpy as jnp
from jax import lax
from jax.experimental import pallas as pl
from jax.experimental.pallas import tpu as pltpu
from jax.experimental.pallas import tpu_sc as plsc

_NC = 2
_NS = 16
_NW = _NC * _NS
_L = 16

_B = 16


def _pick_chunk(epw):
    for cand in range(128, 7, -8):
        if epw % cand == 0:
            return cand
    raise ValueError(f"no valid chunk for {epw}")


_NB = 4
_NB2 = 2 * _NB


@functools.lru_cache(maxsize=None)
def _make_agg(n, d, e):
    assert e % _NW == 0 and n % _NS == 0 and d % _L == 0
    epw = e // _NW
    ch = 40
    chp = ch + (-ch) % _L
    assert epw % (2 * ch) == 0
    nch = epw // ch
    rpt = n // _NS
    zr = 1
    for cand in range(32, 0, -1):
        if rpt % cand == 0:
            zr = cand
            break
    nv = d // _L

    def body(h_hbm, src_hbm, dst_hbm, out_hbm, zbuf, acc,
             rows0, rows1, srcv0, srcv1, dstv0, dstv1, fbuf, fidx,
             gsem0, gsem1, isem0, isem1):
        rows = (rows0, rows1)
        srcv = (srcv0, srcv1)
        dstv = (dstv0, dstv1)
        gsem = (gsem0, gsem1)
        isem = (isem0, isem1)
        c = lax.axis_index("c")
        s = lax.axis_index("s")
        wid = c * _NS + s
        eb = wid * epw

        zrow0 = jnp.zeros((_L,), jnp.float32)
        for rr in range(_L):
            for jj in range(nv):
                plsc.store_scatter(fbuf,
                                   [jnp.full((_L,), rr, jnp.int32),
                                    lax.iota(jnp.int32, _L) + jj * _L], zrow0)

        for rr in range(zr):
            for jj in range(nv):
                zbuf[rr, pl.ds(jj * _L, _L)] = jnp.zeros((_L,), jnp.float32)
        row0 = s * rpt

        @pl.loop(0, rpt // zr)
        def _(i):
            pltpu.sync_copy(zbuf, acc.at[pl.ds(row0 + i * zr, zr), :])

        plsc.subcore_barrier()

        def issue_idx(g, j):
            off = eb + lax.rem(g, nch) * ch
            pltpu.async_copy(src_hbm.at[pl.ds(off, ch)], srcv[j], isem[j])
            pltpu.async_copy(dst_hbm.at[pl.ds(off, ch)], dstv[j].at[pl.ds(0, ch)],
                             isem[j])

        def wait_idx(j):
            pltpu.make_async_copy(src_hbm.at[pl.ds(eb, ch)], srcv[j],
                                  isem[j]).wait()
            pltpu.make_async_copy(dst_hbm.at[pl.ds(eb, ch)],
                                  dstv[j].at[pl.ds(0, ch)], isem[j]).wait()

        def issue_gather(j):
            pltpu.async_copy(h_hbm.at[srcv[j]], rows[j], gsem[j])

        def wait_gather(j):
            pltpu.make_async_copy(h_hbm.at[srcv[j]], rows[j], gsem[j]).wait()

        def flush(prev, acck):
            zrow = jnp.zeros((_L,), jnp.int32)
            colv = lax.iota(jnp.int32, _L)
            for k in range(nv):
                plsc.store_scatter(fbuf, [zrow, colv + k * _L], acck[k])
            fidx[...] = jnp.full((_L,), prev, jnp.int32)
            pltpu.sync_copy(fbuf, acc.at[fidx], add=True)

        def process(j, carry):
            prev, acck = carry
            for i in range(ch):
                if i % _L == 0:
                    dv = dstv[j][pl.ds(i, _L)]
                cur = dv[i % _L]
                ri = [rows[j][i, pl.ds(k * _L, _L)] for k in range(nv)]
                changed = cur != prev

                @pl.when(jnp.logical_and(changed, prev >= 0))
                def _():
                    flush(prev, acck)

                acck = [jnp.where(changed, ri[k], acck[k] + ri[k])
                        for k in range(nv)]
                prev = cur
            return prev, acck

        issue_idx(0, 0)
        wait_idx(0)
        issue_gather(0)
        issue_idx(1, 1)

        init = (jnp.int32(-1),
                [jnp.zeros((_L,), jnp.float32) for _ in range(nv)])

        @pl.loop(0, nch // 2, init_carry=init)
        def carry_loop(i, carry):
            g = 2 * i
            wait_gather(0)
            wait_idx(1)
            issue_gather(1)
            carry = process(0, carry)
            issue_idx(g + 2, 0)
            wait_gather(1)
            wait_idx(0)
            issue_gather(0)
            carry = process(1, carry)
            issue_idx(g + 3, 1)
            return carry

        prev, acck = carry_loop

        @pl.when(prev >= 0)
        def _():
            flush(prev, acck)

        wait_gather(0)
        wait_idx(1)

        plsc.subcore_barrier()
        pltpu.sync_copy(acc.at[pl.ds(row0, rpt), :],
                        out_hbm.at[c, pl.ds(row0, rpt), :])

    return pl.kernel(
        body,
        out_type=jax.ShapeDtypeStruct((_NC, n, d), jnp.float32),
        mesh=plsc.VectorSubcoreMesh(core_axis_name="c", subcore_axis_name="s",
                                    num_cores=_NC, num_subcores=_NS),
        compiler_params=pltpu.CompilerParams(needs_layout_passes=False,
                                             use_tc_tiling_on_sc=False),
        scratch_types=[
            pltpu.VMEM((zr, d), jnp.float32),
            pltpu.VMEM_SHARED((n, d), jnp.float32),
            pltpu.VMEM((ch, d), jnp.float32),
            pltpu.VMEM((ch, d), jnp.float32),
            pltpu.VMEM((ch,), jnp.int32),
            pltpu.VMEM((ch,), jnp.int32),
            pltpu.VMEM((chp,), jnp.int32),
            pltpu.VMEM((chp,), jnp.int32),
            pltpu.VMEM((_L, d), jnp.float32),
            pltpu.VMEM((_L,), jnp.int32),
            pltpu.SemaphoreType.DMA,
            pltpu.SemaphoreType.DMA,
            pltpu.SemaphoreType.DMA,
            pltpu.SemaphoreType.DMA,
        ],
    )


@functools.lru_cache(maxsize=None)
def _make_deg(n, e):
    assert e % _NW == 0 and n % _L == 0
    epw = e // _NW
    ch = _pick_chunk(epw)
    nch = epw // ch

    def body(dst_hbm, out_hbm, dst_all, hist):
        c = lax.axis_index("c")
        s = lax.axis_index("s")
        wid = c * _NS + s

        pltpu.sync_copy(dst_hbm.at[pl.ds(wid * epw, epw)], dst_all)

        @pl.loop(0, n // _L)
        def _(i):
            hist[pl.ds(i * _L, _L)] = jnp.zeros((_L,), jnp.float32)

        ones = jnp.ones((_L,), jnp.float32)

        @pl.loop(0, epw // _L)
        def _(j):
            idx = dst_all[pl.ds(j * _L, _L)]
            plsc.addupdate_scatter(hist, [idx], ones)

        pltpu.sync_copy(hist, out_hbm.at[wid])

    return pl.kernel(
        body,
        out_type=jax.ShapeDtypeStruct((_NW, n), jnp.float32),
        mesh=plsc.VectorSubcoreMesh(core_axis_name="c", subcore_axis_name="s",
                                    num_cores=_NC, num_subcores=_NS),
        compiler_params=pltpu.CompilerParams(needs_layout_passes=False,
                                             use_tc_tiling_on_sc=False),
        scratch_types=[
            pltpu.VMEM((epw,), jnp.int32),
            pltpu.VMEM((n,), jnp.float32),
        ],
    )


def _prep_body(parts_ref, r_ref):
    deg = jnp.sum(parts_ref[...], axis=0)
    r_ref[...] = jnp.maximum(deg, 1.0)[:, None]


@functools.lru_cache(maxsize=None)
def _make_prep(n):
    return pl.pallas_call(
        _prep_body,
        out_shape=jax.ShapeDtypeStruct((n, 1), jnp.float32),
    )


def _dense_body(relu, p_ref, r_ref, wt_ref, b_ref, o_ref):
    h = (p_ref[0] + p_ref[1]) / r_ref[...]
    y = jnp.dot(h, wt_ref[...], preferred_element_type=jnp.float32) + b_ref[...]
    o_ref[...] = jnp.maximum(y, 0.0) if relu else y


@functools.lru_cache(maxsize=None)
def _make_dense(n, d, rb, relu):
    grid = (n // rb,)
    return pl.pallas_call(
        functools.partial(_dense_body, relu),
        grid=grid,
        in_specs=[
            pl.BlockSpec((_NC, rb, d), lambda g: (0, g, 0)),
            pl.BlockSpec((rb, 1), lambda g: (g, 0)),
            pl.BlockSpec((d, d), lambda g: (0, 0)),
            pl.BlockSpec((1, d), lambda g: (0, 0)),
        ],
        out_specs=pl.BlockSpec((rb, d), lambda g: (g, 0)),
        out_shape=jax.ShapeDtypeStruct((n, d), jnp.float32),
    )


def _densepool_body(p_ref, r_ref, wt_ref, b_ref, bid_ref, o_ref):
    g = pl.program_id(0)
    h = (p_ref[0] + p_ref[1]) / r_ref[...]
    y = jnp.dot(h, wt_ref[...], preferred_element_type=jnp.float32) + b_ref[...]

    @pl.when(g == 0)
    def _():
        o_ref[...] = jnp.full(o_ref.shape, -jnp.inf, jnp.float32)

    bid = bid_ref[0]
    for k in range(_B):
        mk = bid == k
        mx = jnp.max(jnp.where(mk, y, -jnp.inf), axis=0)
        o_ref[k, :] = jnp.maximum(o_ref[k, :], mx)


@functools.lru_cache(maxsize=None)
def _make_densepool(n, d, rb):
    nb = n // rb
    return pl.pallas_call(
        _densepool_body,
        grid=(nb,),
        in_specs=[
            pl.BlockSpec((_NC, rb, d), lambda g: (0, g, 0)),
            pl.BlockSpec((rb, 1), lambda g: (g, 0)),
            pl.BlockSpec((d, d), lambda g: (0, 0)),
            pl.BlockSpec((1, d), lambda g: (0, 0)),
            pl.BlockSpec((1, rb, 1), lambda g: (g, 0, 0)),
        ],
        out_specs=pl.BlockSpec((_B, d), lambda g: (0, 0)),
        out_shape=jax.ShapeDtypeStruct((_B, d), jnp.float32),
    )


def _heads_body(pool_ref, w1, b1, w2, b2, wo, bo, v1, c1, v2, c2, vo, co,
                q1_ref, q2_ref):
    d = pool_ref.shape[1]
    p = jnp.concatenate(
        [pool_ref[...], jnp.zeros((d - _B, d), jnp.float32)], axis=0)

    def mlp(wa, ba, wb, bb, wc, bc):
        a = jnp.maximum(jnp.dot(p, wa[...], preferred_element_type=jnp.float32) + ba[...], 0.0)
        a = jnp.maximum(jnp.dot(a, wb[...], preferred_element_type=jnp.float32) + bb[...], 0.0)
        return jnp.dot(a, wc[...], preferred_element_type=jnp.float32) + bc[...]

    q1_ref[...] = mlp(w1, b1, w2, b2, wo, bo)[:_B]
    q2_ref[...] = mlp(v1, c1, v2, c2, vo, co)[:_B]


@functools.lru_cache(maxsize=None)
def _make_heads(d):
    return pl.pallas_call(
        _heads_body,
        out_shape=(jax.ShapeDtypeStruct((_B, 1), jnp.float32),
                   jax.ShapeDtypeStruct((_B, 1), jnp.float32)),
    )


def kernel(x, action, edge_index, batch,
           W_g0, b_g0, W_g1, b_g1, W_g2, b_g2,
           q1_W, q1_b, q1_2_W, q1_2_b, q1_out_W, q1_out_b,
           q2_W, q2_b, q2_2_W, q2_2_b, q2_out_W, q2_out_b):
    n = x.shape[0]
    e = edge_index.shape[1]
    d = W_g0.shape[0]
    order = jnp.argsort(edge_index[1], stable=True)
    src = edge_index[0][order]
    dst = edge_index[1][order]
    h0 = jnp.concatenate([x, action], axis=1)

    agg = _make_agg(n, d, e)
    rb = 1000
    dense_r = _make_dense(n, d, rb, True)

    degp = _make_deg(n, e)(dst)
    r = _make_prep(n)(degp)

    p = agg(h0, src, dst)
    h1 = dense_r(p, r, W_g0.T, b_g0.reshape(1, d))
    p = agg(h1, src, dst)
    h2 = dense_r(p, r, W_g1.T, b_g1.reshape(1, d))
    p = agg(h2, src, dst)
    pooled = _make_densepool(n, d, rb)(
        p, r, W_g2.T, b_g2.reshape(1, d), batch.reshape(n // rb, rb, 1))

    q1, q2 = _make_heads(d)(
        pooled,
        q1_W.T, q1_b.reshape(1, d), q1_2_W.T, q1_2_b.reshape(1, d),
        q1_out_W.T, q1_out_b.reshape(1, 1),
        q2_W.T, q2_b.reshape(1, d), q2_2_W.T, q2_2_b.reshape(1, d),
        q2_out_W.T, q2_out_b.reshape(1, 1))
    return (q1, q2)
